# Initial kernel scaffold; baseline (speedup 1.0000x reference)
import jax
import jax.numpy as jnp
from jax.experimental import pallas as pl


def kernel(features, memory_bank):
    raise NotImplementedError("write your pallas kernel here")


# speedup vs baseline: 4.4213x; 4.4213x over previous
import numpy as np
import jax
import jax.numpy as jnp
from jax.experimental import pallas as pl
from jax.experimental.pallas import tpu as pltpu

_GRID_HW = 16
_IMG_HW = 224
_BLK = 2000


def _bilinear_matrix(out_size: int, in_size: int) -> np.ndarray:
    scale = in_size / out_size
    mat = np.zeros((out_size, in_size), dtype=np.float64)
    for i in range(out_size):
        x = (i + 0.5) * scale - 0.5
        j0 = int(np.floor(x))
        t = x - j0
        for j, w in ((j0, 1.0 - t), (j0 + 1, t)):
            if w > 0.0:
                mat[i, min(max(j, 0), in_size - 1)] += w
    return mat.astype(np.float32)


def _knn_kernel(q_ref, bank_ref, rmat_ref, score_ref, amap_ref,
                qn_ref, maxcos_ref):
    i = pl.program_id(0)
    nblk = pl.num_programs(0)

    @pl.when(i == 0)
    def _init():
        q = q_ref[...]
        qn = q / (jnp.sqrt(jnp.sum(q * q, axis=1, keepdims=True)) + 1e-8)
        qn_ref[...] = qn.astype(jnp.bfloat16)
        maxcos_ref[...] = jnp.full(maxcos_ref.shape, -jnp.inf, jnp.float32)

    b = bank_ref[...]
    inv_bn = jax.lax.rsqrt(jnp.sum(b * b, axis=1, keepdims=True))
    s = jax.lax.dot_general(
        b.astype(jnp.bfloat16), qn_ref[...],
        dimension_numbers=(((1,), (1,)), ((), ())),
        preferred_element_type=jnp.float32)
    s = s * inv_bn
    m = jnp.max(s, axis=0, keepdims=True)
    maxcos_ref[...] = jnp.maximum(maxcos_ref[...], m)

    @pl.when(i == nblk - 1)
    def _finish():
        mc = maxcos_ref[...]
        dist = jnp.sqrt(jnp.maximum(2.0 - 2.0 * mc, 1e-12))
        n = dist.shape[1]
        m1 = jnp.max(dist, axis=1, keepdims=True)
        lane = jax.lax.broadcasted_iota(jnp.int32, dist.shape, 1)
        first = jnp.min(jnp.where(dist == m1, lane, n), axis=1, keepdims=True)
        m2 = jnp.max(jnp.where(lane == first, -jnp.inf, dist),
                     axis=1, keepdims=True)
        score_ref[...] = 0.5 * (m1 + m2)
        a = jnp.concatenate(
            [dist[:, _GRID_HW * r:_GRID_HW * (r + 1)] for r in range(_GRID_HW)],
            axis=0)
        r = rmat_ref[...]
        up = jnp.dot(jnp.dot(r, a, preferred_element_type=jnp.float32), r.T,
                     preferred_element_type=jnp.float32)
        amap_ref[...] = up.reshape(1, 1, _IMG_HW, _IMG_HW)


def kernel(features, memory_bank):
    bsz, n, d = features.shape
    m = memory_bank.shape[0]
    q = features.reshape(bsz * n, d)
    rmat = jnp.asarray(_bilinear_matrix(_IMG_HW, _GRID_HW))
    nblk = m // _BLK

    score, amap = pl.pallas_call(
        _knn_kernel,
        grid=(nblk,),
        in_specs=[
            pl.BlockSpec((bsz * n, d), lambda i: (0, 0)),
            pl.BlockSpec((_BLK, d), lambda i: (i, 0)),
            pl.BlockSpec((_IMG_HW, _GRID_HW), lambda i: (0, 0)),
        ],
        out_specs=[
            pl.BlockSpec((1, 1), lambda i: (0, 0)),
            pl.BlockSpec((1, 1, _IMG_HW, _IMG_HW), lambda i: (0, 0, 0, 0)),
        ],
        out_shape=[
            jax.ShapeDtypeStruct((1, 1), jnp.float32),
            jax.ShapeDtypeStruct((1, 1, _IMG_HW, _IMG_HW), jnp.float32),
        ],
        scratch_shapes=[
            pltpu.VMEM((bsz * n, d), jnp.bfloat16),
            pltpu.VMEM((1, bsz * n), jnp.float32),
        ],
    )(q, memory_bank, rmat)
    return score, amap


# speedup vs baseline: 5.7052x; 1.2904x over previous
import numpy as np
import jax
import jax.numpy as jnp
from jax.experimental import pallas as pl
from jax.experimental.pallas import tpu as pltpu

_GRID_HW = 16
_IMG_HW = 224
_BLK = 4000


def _bilinear_matrix(out_size: int, in_size: int) -> np.ndarray:
    scale = in_size / out_size
    mat = np.zeros((out_size, in_size), dtype=np.float64)
    for i in range(out_size):
        x = (i + 0.5) * scale - 0.5
        j0 = int(np.floor(x))
        t = x - j0
        for j, w in ((j0, 1.0 - t), (j0 + 1, t)):
            if w > 0.0:
                mat[i, min(max(j, 0), in_size - 1)] += w
    return mat.astype(np.float32)


def _knn_kernel(q_ref, bank_ref, rmat_ref, score_ref, amap_ref,
                qn_ref, maxcos_ref):
    i = pl.program_id(0)
    nblk = pl.num_programs(0)

    @pl.when(i == 0)
    def _init():
        q = q_ref[...]
        qn = q / (jnp.sqrt(jnp.sum(q * q, axis=1, keepdims=True)) + 1e-8)
        qn_ref[...] = qn.astype(jnp.bfloat16)
        maxcos_ref[...] = jnp.full(maxcos_ref.shape, -jnp.inf, jnp.float32)

    b = bank_ref[...]
    inv_bn = jax.lax.rsqrt(jnp.sum(b * b, axis=1, keepdims=True))
    s = jax.lax.dot_general(
        b.astype(jnp.bfloat16), qn_ref[...],
        dimension_numbers=(((1,), (1,)), ((), ())),
        preferred_element_type=jnp.float32)
    s = s * inv_bn
    m = jnp.max(s, axis=0, keepdims=True)
    maxcos_ref[...] = jnp.maximum(maxcos_ref[...], m)

    @pl.when(i == nblk - 1)
    def _finish():
        mc = maxcos_ref[...]
        dist = jnp.sqrt(jnp.maximum(2.0 - 2.0 * mc, 1e-12))
        n = dist.shape[1]
        m1 = jnp.max(dist, axis=1, keepdims=True)
        lane = jax.lax.broadcasted_iota(jnp.int32, dist.shape, 1)
        first = jnp.min(jnp.where(dist == m1, lane, n), axis=1, keepdims=True)
        m2 = jnp.max(jnp.where(lane == first, -jnp.inf, dist),
                     axis=1, keepdims=True)
        score_ref[...] = 0.5 * (m1 + m2)
        a = jnp.concatenate(
            [dist[:, _GRID_HW * r:_GRID_HW * (r + 1)] for r in range(_GRID_HW)],
            axis=0)
        r = rmat_ref[...]
        up = jnp.dot(jnp.dot(r, a, preferred_element_type=jnp.float32), r.T,
                     preferred_element_type=jnp.float32)
        amap_ref[...] = up.reshape(1, 1, _IMG_HW, _IMG_HW)


def kernel(features, memory_bank):
    bsz, n, d = features.shape
    m = memory_bank.shape[0]
    q = features.reshape(bsz * n, d)
    rmat = jnp.asarray(_bilinear_matrix(_IMG_HW, _GRID_HW))
    nblk = m // _BLK

    score, amap = pl.pallas_call(
        _knn_kernel,
        grid=(nblk,),
        in_specs=[
            pl.BlockSpec((bsz * n, d), lambda i: (0, 0)),
            pl.BlockSpec((_BLK, d), lambda i: (i, 0)),
            pl.BlockSpec((_IMG_HW, _GRID_HW), lambda i: (0, 0)),
        ],
        out_specs=[
            pl.BlockSpec((1, 1), lambda i: (0, 0)),
            pl.BlockSpec((1, 1, _IMG_HW, _IMG_HW), lambda i: (0, 0, 0, 0)),
        ],
        out_shape=[
            jax.ShapeDtypeStruct((1, 1), jnp.float32),
            jax.ShapeDtypeStruct((1, 1, _IMG_HW, _IMG_HW), jnp.float32),
        ],
        scratch_shapes=[
            pltpu.VMEM((bsz * n, d), jnp.bfloat16),
            pltpu.VMEM((1, bsz * n), jnp.float32),
        ],
    )(q, memory_bank, rmat)
    return score, amap


# speedup vs baseline: 6.5882x; 1.1548x over previous
import numpy as np
import jax
import jax.numpy as jnp
from jax.experimental import pallas as pl
from jax.experimental.pallas import tpu as pltpu

_GRID_HW = 16
_IMG_HW = 224
_BLK = 8000


def _bilinear_matrix(out_size: int, in_size: int) -> np.ndarray:
    scale = in_size / out_size
    mat = np.zeros((out_size, in_size), dtype=np.float64)
    for i in range(out_size):
        x = (i + 0.5) * scale - 0.5
        j0 = int(np.floor(x))
        t = x - j0
        for j, w in ((j0, 1.0 - t), (j0 + 1, t)):
            if w > 0.0:
                mat[i, min(max(j, 0), in_size - 1)] += w
    return mat.astype(np.float32)


def _knn_kernel(q_ref, bank_ref, rmat_ref, score_ref, amap_ref,
                qn_ref, maxcos_ref):
    i = pl.program_id(0)
    nblk = pl.num_programs(0)

    @pl.when(i == 0)
    def _init():
        q = q_ref[...]
        qn = q / (jnp.sqrt(jnp.sum(q * q, axis=1, keepdims=True)) + 1e-8)
        qn_ref[...] = qn.astype(jnp.bfloat16)
        maxcos_ref[...] = jnp.full(maxcos_ref.shape, -jnp.inf, jnp.float32)

    b = bank_ref[...]
    inv_bn = jax.lax.rsqrt(jnp.sum(b * b, axis=1, keepdims=True))
    s = jax.lax.dot_general(
        b.astype(jnp.bfloat16), qn_ref[...],
        dimension_numbers=(((1,), (1,)), ((), ())),
        preferred_element_type=jnp.float32)
    s = s * inv_bn
    m = jnp.max(s, axis=0, keepdims=True)
    maxcos_ref[...] = jnp.maximum(maxcos_ref[...], m)

    @pl.when(i == nblk - 1)
    def _finish():
        mc = maxcos_ref[...]
        dist = jnp.sqrt(jnp.maximum(2.0 - 2.0 * mc, 1e-12))
        n = dist.shape[1]
        m1 = jnp.max(dist, axis=1, keepdims=True)
        lane = jax.lax.broadcasted_iota(jnp.int32, dist.shape, 1)
        first = jnp.min(jnp.where(dist == m1, lane, n), axis=1, keepdims=True)
        m2 = jnp.max(jnp.where(lane == first, -jnp.inf, dist),
                     axis=1, keepdims=True)
        score_ref[...] = 0.5 * (m1 + m2)
        a = jnp.concatenate(
            [dist[:, _GRID_HW * r:_GRID_HW * (r + 1)] for r in range(_GRID_HW)],
            axis=0)
        r = rmat_ref[...]
        up = jnp.dot(jnp.dot(r, a, preferred_element_type=jnp.float32), r.T,
                     preferred_element_type=jnp.float32)
        amap_ref[...] = up.reshape(1, 1, _IMG_HW, _IMG_HW)


def kernel(features, memory_bank):
    bsz, n, d = features.shape
    m = memory_bank.shape[0]
    q = features.reshape(bsz * n, d)
    rmat = jnp.asarray(_bilinear_matrix(_IMG_HW, _GRID_HW))
    nblk = m // _BLK

    score, amap = pl.pallas_call(
        _knn_kernel,
        grid=(nblk,),
        in_specs=[
            pl.BlockSpec((bsz * n, d), lambda i: (0, 0)),
            pl.BlockSpec((_BLK, d), lambda i: (i, 0)),
            pl.BlockSpec((_IMG_HW, _GRID_HW), lambda i: (0, 0)),
        ],
        out_specs=[
            pl.BlockSpec((1, 1), lambda i: (0, 0)),
            pl.BlockSpec((1, 1, _IMG_HW, _IMG_HW), lambda i: (0, 0, 0, 0)),
        ],
        out_shape=[
            jax.ShapeDtypeStruct((1, 1), jnp.float32),
            jax.ShapeDtypeStruct((1, 1, _IMG_HW, _IMG_HW), jnp.float32),
        ],
        scratch_shapes=[
            pltpu.VMEM((bsz * n, d), jnp.bfloat16),
            pltpu.VMEM((1, bsz * n), jnp.float32),
        ],
    )(q, memory_bank, rmat)
    return score, amap
